# Initial kernel scaffold; baseline (speedup 1.0000x reference)
import jax
import jax.numpy as jnp
from jax.experimental import pallas as pl


def kernel(x, edge_index, segment_ids, y, W1, b1, W2, b2, down_k, Wq, Wk, Wv, Wo, Wfc, bfc, Wrc, brc, conv_w, conv_b, Wcls, bcls):
    raise NotImplementedError("write your pallas kernel here")


# speedup vs baseline: 1.0881x; 1.0881x over previous
import functools

import jax
import jax.numpy as jnp
from jax import lax
from jax.experimental import pallas as pl
from jax.experimental.pallas import tpu as pltpu
from jax.experimental.pallas import tpu_sc as plsc

N = 10000
NP = 10240
D = 128
E = 320000
EP = 327680
G = 64
K = 20
H = 4
DH = D // H
NC = 2
NS = 16
NW = NC * NS
EPT = EP // NW
ECH = 128
NECH = EPT // ECH
RPT = NP // NW
RCH = 80
ZR = 8

_HIGH = jax.lax.Precision.DEFAULT


def _dot(a, b, dims):
    return lax.dot_general(a, b, (dims, ((), ())), precision=_HIGH,
                           preferred_element_type=jnp.float32)


def _sc_mesh():
    return plsc.VectorSubcoreMesh(core_axis_name="c", subcore_axis_name="s")


def _edge_agg_sc(x_ext, srcp, dstp):
    @functools.partial(
        pl.kernel,
        out_type=jax.ShapeDtypeStruct((NC, NP, D), jnp.float32),
        mesh=_sc_mesh(),
        scratch_types=[
            pltpu.VMEM((1, ECH), jnp.int32),
            pltpu.VMEM((1, ECH), jnp.int32),
            pltpu.VMEM((ECH, D), jnp.float32),
            pltpu.VMEM((ZR, D), jnp.float32),
            pltpu.VMEM_SHARED((NP, D), jnp.float32),
            pltpu.SemaphoreType.DMA,
        ],
    )
    def k(x_hbm, src_hbm, dst_hbm, out_hbm, sidx, didx, rows, zbuf, acc, sem):
        c = lax.axis_index("c")
        s = lax.axis_index("s")
        wid = c * NS + s

        @pl.loop(0, ZR)
        def _(r):
            @pl.loop(0, D, step=16)
            def _(j):
                zbuf[r, pl.ds(j, 16)] = jnp.zeros((16,), jnp.float32)

        rows_per_sub = NP // NS

        @pl.loop(0, rows_per_sub, step=ZR)
        def _(r):
            pltpu.sync_copy(zbuf, acc.at[pl.ds(s * rows_per_sub + r, ZR)])

        plsc.subcore_barrier()

        base = wid * EPT

        @pl.loop(0, NECH)
        def _(it):
            off = base + it * ECH
            pltpu.sync_copy(src_hbm.at[pl.ds(off, ECH)], sidx.at[0])
            pltpu.sync_copy(dst_hbm.at[pl.ds(off, ECH)], didx.at[0])
            pltpu.async_copy(x_hbm.at[sidx.at[0]], rows, sem).wait()
            pltpu.sync_copy(rows, acc.at[didx.at[0]], add=True)

        plsc.subcore_barrier()
        pltpu.sync_copy(acc.at[pl.ds(s * rows_per_sub, rows_per_sub)],
                        out_hbm.at[c, pl.ds(s * rows_per_sub, rows_per_sub)])

    return k(x_ext, srcp, dstp)


def _scatter_rows_sc(vals, idx):
    @functools.partial(
        pl.kernel,
        out_type=jax.ShapeDtypeStruct((NP, D), jnp.float32),
        mesh=_sc_mesh(),
        scratch_types=[
            pltpu.VMEM((1, RCH), jnp.int32),
            pltpu.VMEM((RCH, D), jnp.float32),
        ],
    )
    def k(v_hbm, i_hbm, out_hbm, ib, rows):
        c = lax.axis_index("c")
        s = lax.axis_index("s")
        wid = c * NS + s

        @pl.loop(0, RPT // RCH)
        def _(it):
            off = wid * RPT + it * RCH
            pltpu.sync_copy(i_hbm.at[pl.ds(off, RCH)], ib.at[0])
            pltpu.sync_copy(v_hbm.at[pl.ds(off, RCH)], rows)
            pltpu.sync_copy(rows, out_hbm.at[ib.at[0]])

    return k(vals, idx)


def _gather_rows_sc(vals, idx):
    @functools.partial(
        pl.kernel,
        out_type=jax.ShapeDtypeStruct((NP, D), jnp.float32),
        mesh=_sc_mesh(),
        scratch_types=[
            pltpu.VMEM((1, RCH), jnp.int32),
            pltpu.VMEM((RCH, D), jnp.float32),
            pltpu.SemaphoreType.DMA,
        ],
    )
    def k(v_hbm, i_hbm, out_hbm, ib, rows, sem):
        c = lax.axis_index("c")
        s = lax.axis_index("s")
        wid = c * NS + s

        @pl.loop(0, RPT // RCH)
        def _(it):
            off = wid * RPT + it * RCH
            pltpu.sync_copy(i_hbm.at[pl.ds(off, RCH)], ib.at[0])
            pltpu.async_copy(v_hbm.at[ib.at[0]], rows, sem).wait()
            pltpu.sync_copy(rows, out_hbm.at[pl.ds(off, RCH)])

    return k(vals, idx)


def _phase1_body(x_ref, aggp_ref, w1_ref, b1_ref, seg_ref, dk_ref, wq_ref,
                 wk_ref, wv_ref, wo_ref, wfc_ref, bfc_ref, wrc_ref, brc_ref,
                 h_ref, score_ref, feats_ref):
    xs = x_ref[...] + aggp_ref[0] + aggp_ref[1]
    h = jnp.maximum(_dot(xs, w1_ref[...], ((1,), (0,))) + b1_ref[...], 0.0)
    row = lax.broadcasted_iota(jnp.int32, (NP, 1), 0)
    h = jnp.where(row < N, h, 0.0)
    h_ref[...] = h

    seg = seg_ref[...]
    gi = lax.broadcasted_iota(jnp.int32, (NP, G), 1)
    onehot = (seg == gi).astype(jnp.float32)
    pooled_sum = _dot(onehot, h, ((0,), (0,)))
    counts = jnp.sum(onehot, axis=0, keepdims=True)
    inv = 1.0 / jnp.maximum(counts, 1.0)
    pooled = pooled_sum * inv.reshape(G, 1)

    q = _dot(pooled, wq_ref[...], ((1,), (0,)))
    kk = _dot(dk_ref[...], wk_ref[...], ((1,), (0,)))
    vv = _dot(dk_ref[...], wv_ref[...], ((1,), (0,)))
    outs = []
    scale = 1.0 / (DH ** 0.5)
    for hh in range(H):
        sl = slice(hh * DH, (hh + 1) * DH)
        logits = _dot(q[:, sl], kk[:, sl], ((1,), (1,))) * scale
        m = jnp.max(logits, axis=1, keepdims=True)
        ex = jnp.exp(logits - m)
        attn = ex / jnp.sum(ex, axis=1, keepdims=True)
        outs.append(_dot(attn, vv[:, sl], ((1,), (0,))))
    feats = _dot(jnp.concatenate(outs, axis=1), wo_ref[...], ((1,), (0,)))
    z = jnp.tanh(jnp.mean(feats, axis=0, keepdims=True))
    feats = feats + _dot(feats + z, wfc_ref[...], ((1,), (0,))) + bfc_ref[...]
    feats_ref[...] = feats

    tail = _dot(z, wrc_ref[...], ((1,), (0,))) + brc_ref[...]
    score = _dot(h, tail, ((1,), (1,)))
    score_ref[...] = jnp.where(row < N, score, jnp.inf)


def _phase1_tc(x_ext, aggp, W1, b1, seg2, down_k, Wq, Wk, Wv, Wo, Wfc, bfc,
               Wrc, brc):
    return pl.pallas_call(
        _phase1_body,
        out_shape=[
            jax.ShapeDtypeStruct((NP, D), jnp.float32),
            jax.ShapeDtypeStruct((NP, 1), jnp.float32),
            jax.ShapeDtypeStruct((G, D), jnp.float32),
        ],
    )(x_ext, aggp, W1, b1, seg2, down_k, Wq, Wk, Wv, Wo, Wfc, bfc, Wrc, brc)


IB = 256
NJ = NP // 128


def _rank_body(col_ref, row_ref, rank_ref):
    ib = pl.program_id(0)
    si = col_ref[...]
    iglob = ib * IB + lax.broadcasted_iota(jnp.int32, (IB, 128), 0)

    def body(jc, acc):
        sj = row_ref[:, pl.ds(jc * 128, 128)]
        jglob = jc * 128 + lax.broadcasted_iota(jnp.int32, (IB, 128), 1)
        lt = sj < si
        tie = (sj == si) & (jglob < iglob)
        return acc + jnp.where(lt | tie, 1.0, 0.0)

    acc = lax.fori_loop(0, NJ, body, jnp.zeros((IB, 128), jnp.float32))
    rank_ref[...] = jnp.sum(acc, axis=1, keepdims=True).astype(jnp.int32)


def _rank_tc(score_col, score_row):
    return pl.pallas_call(
        _rank_body,
        grid=(NP // IB,),
        in_specs=[
            pl.BlockSpec((IB, 1), lambda i: (i, 0)),
            pl.BlockSpec((1, NP), lambda i: (0, 0)),
        ],
        out_specs=pl.BlockSpec((IB, 1), lambda i: (i, 0)),
        out_shape=jax.ShapeDtypeStruct((NP, 1), jnp.int32),
    )(score_col, score_row)


def _conv_body(sh_ref, w0_ref, w1_ref, w2_ref, b_ref, out_ref):
    sh = sh_ref[...]
    c0 = _dot(sh, w0_ref[...], ((1,), (0,)))
    c1 = _dot(sh, w1_ref[...], ((1,), (0,)))
    c2 = _dot(sh, w2_ref[...], ((1,), (0,)))
    zero = jnp.zeros((1, D), jnp.float32)
    sd = jnp.concatenate([zero, c0[:-1]], axis=0)
    su = jnp.concatenate([c2[1:], zero], axis=0)
    out_ref[...] = c1 + sd + su + b_ref[...]


def _conv_tc(SH, W0, W1c, W2c, cb):
    return pl.pallas_call(
        _conv_body,
        out_shape=jax.ShapeDtypeStruct((NP, D), jnp.float32),
    )(SH, W0, W1c, W2c, cb)


def _phase2_body(r_ref, aggp_ref, w2_ref, b2_ref, seg_ref, feats_ref,
                 wcls_ref, bcls_ref, y_ref, loss_ref, logits_ref):
    rs = r_ref[...] + aggp_ref[0] + aggp_ref[1]
    out = jnp.maximum(_dot(rs, w2_ref[...], ((1,), (0,))) + b2_ref[...], 0.0)

    seg = seg_ref[...]
    gi = lax.broadcasted_iota(jnp.int32, (NP, G), 1)
    onehot = (seg == gi).astype(jnp.float32)
    pooled_sum = _dot(onehot, out, ((0,), (0,)))
    counts = jnp.sum(onehot, axis=0, keepdims=True)
    inv = 1.0 / jnp.maximum(counts, 1.0)
    out_g = pooled_sum * inv.reshape(G, 1)

    lp = _dot(out_g + feats_ref[...], wcls_ref[...], ((1,), (0,))) + bcls_ref[...]
    m = jnp.max(lp, axis=1, keepdims=True)
    ex = jnp.exp(lp - m)
    logits = lp - m - jnp.log(jnp.sum(ex, axis=1, keepdims=True))
    logits_ref[...] = logits

    y = y_ref[...]
    yf = y.astype(jnp.float32)
    n_pos = jnp.maximum(jnp.sum(yf), 1.0)
    n_neg = jnp.maximum(jnp.sum(1.0 - yf), 1.0)
    sw = jnp.where(y == 1, 1.0 / n_pos, 1.0 / n_neg)
    picked = jnp.where(y == 1, logits[:, 1:2], logits[:, 0:1])
    loss_ref[...] = (-jnp.sum(sw * picked) / jnp.sum(sw)).reshape(1, 1)


def _phase2_tc(recnn, agg2p, W2, b2, seg2, feats, Wcls, bcls, y2):
    return pl.pallas_call(
        _phase2_body,
        out_shape=[
            jax.ShapeDtypeStruct((1, 1), jnp.float32),
            jax.ShapeDtypeStruct((G, C2), jnp.float32),
        ],
    )(recnn, agg2p, W2, b2, seg2, feats, Wcls, bcls, y2)


C2 = 2


def kernel(x, edge_index, segment_ids, y, W1, b1, W2, b2, down_k, Wq, Wk, Wv,
           Wo, Wfc, bfc, Wrc, brc, conv_w, conv_b, Wcls, bcls):
    x_ext = jnp.concatenate([x, jnp.zeros((NP - N, D), jnp.float32)], axis=0)
    src = edge_index[0]
    dst = edge_index[1]
    srcp = jnp.concatenate([src, jnp.zeros((EP - E,), jnp.int32)])
    dstp = jnp.concatenate([dst, jnp.full((EP - E,), NP - 1, jnp.int32)])
    seg2 = jnp.concatenate([segment_ids,
                            jnp.full((NP - N,), G, jnp.int32)]).reshape(NP, 1)
    b1r = b1.reshape(1, D)
    b2r = b2.reshape(1, D)
    bfcr = bfc.reshape(1, D)
    brcr = brc.reshape(1, D)
    cbr = conv_b.reshape(1, D)
    bclsr = bcls.reshape(1, C2)
    W0c = conv_w[:, :, 0].T
    W1c = conv_w[:, :, 1].T
    W2c = conv_w[:, :, 2].T
    y2 = y.reshape(G, 1)

    def _edge_agg_dbg(v, s_, d_):
        return jnp.stack([
            jax.ops.segment_sum(v[s_[:EP // 2]], d_[:EP // 2], num_segments=NP),
            jax.ops.segment_sum(v[s_[EP // 2:]], d_[EP // 2:], num_segments=NP)])

    if _DBG["agg_split"]:
        aggp = _edge_agg_dbg(x_ext, srcp, dstp)
    elif _DBG["agg_sorted"]:
        order = jnp.argsort(dst, stable=True)
        a_ = jax.ops.segment_sum(x[src[order]], dst[order], num_segments=N)
        aggp = jnp.stack([
            jnp.concatenate([a_, jnp.zeros((NP - N, D), jnp.float32)]),
            jnp.zeros((NP, D), jnp.float32)])
    else:
        a_ = jax.ops.segment_sum(x[src], dst, num_segments=N)
        aggp = jnp.stack([
            jnp.concatenate([a_, jnp.zeros((NP - N, D), jnp.float32)]),
            jnp.zeros((NP, D), jnp.float32)])

    if _DBG["phase1"]:
        h_ext, score, feats = _phase1_tc(x_ext, aggp, W1, b1r, seg2, down_k,
                                         Wq, Wk, Wv, Wo, Wfc, bfcr, Wrc, brcr)
    elif _DBG["hmm"]:
        def _hk(xs_ref, w_ref, b_ref, o_ref):
            o_ref[...] = jnp.maximum(
                _dot(xs_ref[...], w_ref[...], ((1,), (0,))) + b_ref[...], 0.0)
        agg = (aggp[0] + aggp[1])[:N]
        h = pl.pallas_call(
            _hk, out_shape=jax.ShapeDtypeStruct((N, D), jnp.float32))(
                x + agg, W1, b1r)
        counts0 = jax.ops.segment_sum(jnp.ones((N, 1), jnp.float32),
                                      segment_ids, num_segments=G)
        pooled = jax.ops.segment_sum(h, segment_ids, num_segments=G) \
            / jnp.maximum(counts0, 1.0)
        q = (pooled @ Wq).reshape(G, H, DH)
        kq = (down_k @ Wk).reshape(K, H, DH)
        vq = (down_k @ Wv).reshape(K, H, DH)
        attn = jax.nn.softmax(jnp.einsum('ghd,khd->ghk', q, kq)
                              / jnp.sqrt(float(DH)), axis=-1)
        feats = jnp.einsum('ghk,khd->ghd', attn, vq).reshape(G, D) @ Wo
        z = jnp.tanh(jnp.mean(feats, axis=0))
        feats = feats + ((feats + z) @ Wfc + bfc)
        tail = z @ Wrc + brc
        if _DBG["score_pl"]:
            def _sk(h_ref, t_ref, o_ref):
                o_ref[...] = _dot(h_ref[...], t_ref[...], ((1,), (1,)))
            sc_ = pl.pallas_call(
                _sk, out_shape=jax.ShapeDtypeStruct((N, 1), jnp.float32))(
                    h, tail.reshape(1, D)).reshape(N)
        else:
            sc_ = h @ tail
        h_ext = jnp.concatenate([h, jnp.zeros((NP - N, D), jnp.float32)])
        score = jnp.concatenate([sc_, jnp.full((NP - N,), jnp.inf)]) \
            .reshape(NP, 1)
    else:
        agg = (aggp[0] + aggp[1])[:N]
        h = jax.nn.relu((x + agg) @ W1 + b1)
        counts0 = jax.ops.segment_sum(jnp.ones((N, 1), jnp.float32),
                                      segment_ids, num_segments=G)
        pooled = jax.ops.segment_sum(h, segment_ids, num_segments=G) \
            / jnp.maximum(counts0, 1.0)
        q = (pooled @ Wq).reshape(G, H, DH)
        kq = (down_k @ Wk).reshape(K, H, DH)
        vq = (down_k @ Wv).reshape(K, H, DH)
        attn = jax.nn.softmax(jnp.einsum('ghd,khd->ghk', q, kq)
                              / jnp.sqrt(float(DH)), axis=-1)
        feats = jnp.einsum('ghk,khd->ghd', attn, vq).reshape(G, D) @ Wo
        z = jnp.tanh(jnp.mean(feats, axis=0))
        feats = feats + ((feats + z) @ Wfc + bfc)
        tail = z @ Wrc + brc
        sc_ = h @ tail
        h_ext = jnp.concatenate([h, jnp.zeros((NP - N, D), jnp.float32)])
        score = jnp.concatenate([sc_, jnp.full((NP - N,), jnp.inf)]) \
            .reshape(NP, 1)

    if _DBG["rank"]:
        rank = _rank_tc(score, score.reshape(1, NP)).reshape(NP)
    else:
        indi = jnp.argsort(score.reshape(NP))
        rank = jnp.zeros((NP,), jnp.int32).at[indi].set(
            jnp.arange(NP, dtype=jnp.int32))

    if _DBG["sc_perm"]:
        SH = _scatter_rows_sc(h_ext, rank)
    else:
        SH = jnp.zeros((NP, D), jnp.float32).at[rank].set(h_ext)

    if _DBG["conv"]:
        rs2 = _conv_tc(SH, W0c, W1c, W2c, cbr)
    else:
        c0 = SH @ W0c
        c1 = SH @ W1c
        c2 = SH @ W2c
        zrow = jnp.zeros((1, D), jnp.float32)
        rs2 = c1 + jnp.concatenate([zrow, c0[:-1]]) \
            + jnp.concatenate([c2[1:], zrow]) + conv_b

    if _DBG["sc_perm"]:
        recnn = _gather_rows_sc(rs2, rank)
    else:
        recnn = rs2[rank]
    if _DBG["sc_agg2"]:
        agg2p = _edge_agg_sc(recnn, srcp, dstp)
    else:
        a2_ = jax.ops.segment_sum(recnn[:N][src], dst, num_segments=N)
        agg2p = jnp.stack([
            jnp.concatenate([a2_, jnp.zeros((NP - N, D), jnp.float32)]),
            jnp.zeros((NP, D), jnp.float32)])

    if _DBG["phase2"]:
        loss2, logits = _phase2_tc(recnn, agg2p, W2, b2r, seg2, feats, Wcls,
                                   bclsr, y2)
        return (loss2.reshape(()), logits)
    agg2 = (agg2p[0] + agg2p[1])[:N]
    out = jax.nn.relu((recnn[:N] + agg2) @ W2 + b2)
    counts0 = jax.ops.segment_sum(jnp.ones((N, 1), jnp.float32), segment_ids,
                                  num_segments=G)
    out_g = jax.ops.segment_sum(out, segment_ids, num_segments=G) \
        / jnp.maximum(counts0, 1.0)
    logits = jax.nn.log_softmax((out_g + feats) @ Wcls + bcls, axis=1)
    yf = y.astype(jnp.float32)
    n_pos = jnp.maximum(jnp.sum(yf), 1.0)
    n_neg = jnp.maximum(jnp.sum(1.0 - yf), 1.0)
    w = jnp.stack([1.0 / n_neg, 1.0 / n_pos])
    sample_w = w[y]
    picked = jnp.take_along_axis(logits, y[:, None], axis=1)[:, 0]
    loss = -jnp.sum(sample_w * picked) / jnp.sum(sample_w)
    return (loss, logits)


_DBG = {"agg_split": False, "agg_sorted": True, "phase1": False,
        "hmm": True, "score_pl": False, "rank": True, "sc_perm": True,
        "conv": True, "sc_agg2": True, "phase2": True}


# speedup vs baseline: 1.4827x; 1.3627x over previous
import dataclasses
import functools

import jax
import jax.numpy as jnp
from jax import lax
from jax.experimental import pallas as pl
from jax.experimental.pallas import tpu as pltpu
from jax.experimental.pallas import tpu_sc as plsc

N = 10000
NP = 10240
D = 128
E = 320000
EP = 327680
G = 64
K = 20
H = 4
DH = D // H
C2 = 2
NC = 2
NS = 16
NW = NC * NS
EPT = EP // NW
ECH = 128
NECH = EPT // ECH
RPT = NP // NW
RCH = 80
ZR = 8
CAP = EPT + 16
EB = 1024
STG = 16
ARS = RPT + 8

_PREC = jax.lax.Precision.DEFAULT


def _dot(a, b, dims):
    return lax.dot_general(a, b, (dims, ((), ())), precision=_PREC,
                           preferred_element_type=jnp.float32)


def _sc_mesh():
    return plsc.VectorSubcoreMesh(core_axis_name="c", subcore_axis_name="s")


def _sc_params():
    cp = pltpu.CompilerParams()
    if "needs_layout_passes" in pltpu.CompilerParams.__dataclass_fields__:
        cp = dataclasses.replace(cp, needs_layout_passes=False)
    return cp


def _edge_accum_sorted_sc(x_ext, src_s, dst_s, bounds, zrows):
    @functools.partial(
        pl.kernel,
        out_type=jax.ShapeDtypeStruct((NP, D), jnp.float32),
        mesh=_sc_mesh(),
        compiler_params=_sc_params(),
        scratch_types=[
            pltpu.VMEM((80,), jnp.int32),
            pltpu.VMEM((ECH,), jnp.int32),
            pltpu.VMEM((ECH,), jnp.int32),
            pltpu.VMEM((ECH, D), jnp.float32),
            pltpu.VMEM_SHARED((NS * ARS, D), jnp.float32),
            pltpu.SemaphoreType.DMA,
        ],
    )
    def k(x_hbm, src_hbm, dst_hbm, bnd_hbm, z_hbm, out_hbm, bnd, sidx, didx,
          rows, acc, sem):
        c = lax.axis_index("c")
        s = lax.axis_index("s")
        o = c * NS + s
        lo = o * RPT
        sbase = s * ARS

        pltpu.sync_copy(z_hbm, acc.at[pl.ds(sbase, ARS)])
        pltpu.sync_copy(bnd_hbm, bnd)

        st = bnd[pl.ds(o, 16)][0]
        en = bnd[pl.ds(o + 32, 16)][0]
        st0 = pl.multiple_of(st & ~(ECH - 1), ECH)
        nblk = (en - st0 + ECH - 1) // ECH

        def body(b, carry):
            off = st0 + b * ECH
            pltpu.sync_copy(src_hbm.at[pl.ds(off, ECH)], sidx)
            pltpu.sync_copy(dst_hbm.at[pl.ds(off, ECH)], didx)
            for j in range(ECH // 16):
                gpos = off + lax.iota(jnp.int32, 16) + j * 16
                ok = (gpos >= st) & (gpos < en)
                dv = didx[pl.ds(j * 16, 16)]
                sv = sidx[pl.ds(j * 16, 16)]
                didx[pl.ds(j * 16, 16)] = jnp.where(
                    ok, dv - lo + sbase, sbase + RPT)
                sidx[pl.ds(j * 16, 16)] = jnp.where(ok, sv, 0)
            pltpu.async_copy(x_hbm.at[sidx], rows, sem).wait()
            pltpu.sync_copy(rows, acc.at[didx], add=True)
            return carry

        lax.fori_loop(0, nblk, body, 0)

        pltpu.sync_copy(acc.at[pl.ds(sbase, RPT)], out_hbm.at[pl.ds(lo, RPT)])

    return k(x_ext, src_s, dst_s, bounds, zrows)


def _agg1_sc(x_ext, srcp, dstp):
    order = jnp.argsort(dstp, stable=True)
    src_s = jnp.concatenate([srcp[order], jnp.zeros((ECH,), jnp.int32)])
    dst_s = jnp.concatenate([dstp[order],
                             jnp.full((ECH,), NP - 1, jnp.int32)])
    edges = jnp.arange(0, NP + 1, RPT, dtype=jnp.int32)
    b_all = jnp.searchsorted(dst_s[:EP], edges).astype(jnp.int32)
    bounds = jnp.concatenate([b_all[:NW], b_all[1:NW + 1],
                              jnp.zeros((16,), jnp.int32)])
    zrows = jnp.zeros((ARS, D), jnp.float32)
    return _edge_accum_sorted_sc(x_ext, src_s, dst_s, bounds, zrows)


def _edge_agg_sc(x_ext, srcp, dstp):
    @functools.partial(
        pl.kernel,
        out_type=jax.ShapeDtypeStruct((NC, NP, D), jnp.float32),
        mesh=_sc_mesh(),
        scratch_types=[
            pltpu.VMEM((1, ECH), jnp.int32),
            pltpu.VMEM((1, ECH), jnp.int32),
            pltpu.VMEM((ECH, D), jnp.float32),
            pltpu.VMEM((ZR, D), jnp.float32),
            pltpu.VMEM_SHARED((NP, D), jnp.float32),
            pltpu.SemaphoreType.DMA,
        ],
    )
    def k(x_hbm, src_hbm, dst_hbm, out_hbm, sidx, didx, rows, zbuf, acc, sem):
        c = lax.axis_index("c")
        s = lax.axis_index("s")
        wid = c * NS + s
        rows_per_sub = NP // NS

        @pl.loop(0, ZR)
        def _(r):
            @pl.loop(0, D, step=16)
            def _(j):
                zbuf[r, pl.ds(j, 16)] = jnp.zeros((16,), jnp.float32)

        @pl.loop(0, rows_per_sub, step=ZR)
        def _(r):
            pltpu.sync_copy(zbuf, acc.at[pl.ds(s * rows_per_sub + r, ZR)])

        plsc.subcore_barrier()

        base = wid * EPT

        @pl.loop(0, NECH)
        def _(it):
            off = base + it * ECH
            pltpu.sync_copy(src_hbm.at[pl.ds(off, ECH)], sidx.at[0])
            pltpu.sync_copy(dst_hbm.at[pl.ds(off, ECH)], didx.at[0])
            pltpu.async_copy(x_hbm.at[sidx.at[0]], rows, sem).wait()
            pltpu.sync_copy(rows, acc.at[didx.at[0]], add=True)

        plsc.subcore_barrier()
        pltpu.sync_copy(acc.at[pl.ds(s * rows_per_sub, rows_per_sub)],
                        out_hbm.at[c, pl.ds(s * rows_per_sub, rows_per_sub)])

    return k(x_ext, srcp, dstp)


def _scatter_rows_sc(vals, idx):
    @functools.partial(
        pl.kernel,
        out_type=jax.ShapeDtypeStruct((NP, D), jnp.float32),
        mesh=_sc_mesh(),
        scratch_types=[
            pltpu.VMEM((1, RCH), jnp.int32),
            pltpu.VMEM((RCH, D), jnp.float32),
        ],
    )
    def k(v_hbm, i_hbm, out_hbm, ib, rows):
        c = lax.axis_index("c")
        s = lax.axis_index("s")
        wid = c * NS + s

        @pl.loop(0, RPT // RCH)
        def _(it):
            off = wid * RPT + it * RCH
            pltpu.sync_copy(i_hbm.at[pl.ds(off, RCH)], ib.at[0])
            pltpu.sync_copy(v_hbm.at[pl.ds(off, RCH)], rows)
            pltpu.sync_copy(rows, out_hbm.at[ib.at[0]])

    return k(vals, idx)


def _gather_rows_sc(vals, idx):
    @functools.partial(
        pl.kernel,
        out_type=jax.ShapeDtypeStruct((NP, D), jnp.float32),
        mesh=_sc_mesh(),
        scratch_types=[
            pltpu.VMEM((1, RCH), jnp.int32),
            pltpu.VMEM((RCH, D), jnp.float32),
            pltpu.SemaphoreType.DMA,
        ],
    )
    def k(v_hbm, i_hbm, out_hbm, ib, rows, sem):
        c = lax.axis_index("c")
        s = lax.axis_index("s")
        wid = c * NS + s

        @pl.loop(0, RPT // RCH)
        def _(it):
            off = wid * RPT + it * RCH
            pltpu.sync_copy(i_hbm.at[pl.ds(off, RCH)], ib.at[0])
            pltpu.async_copy(v_hbm.at[ib.at[0]], rows, sem).wait()
            pltpu.sync_copy(rows, out_hbm.at[pl.ds(off, RCH)])

    return k(vals, idx)


def _h_body(xs_ref, w_ref, b_ref, o_ref):
    o_ref[...] = jnp.maximum(
        _dot(xs_ref[...], w_ref[...], ((1,), (0,))) + b_ref[...], 0.0)


def _h_tc(xs, W1, b1r):
    return pl.pallas_call(
        _h_body, out_shape=jax.ShapeDtypeStruct((N, D), jnp.float32))(
            xs, W1, b1r)


IB = 256
NJ = NP // 128


def _rank_body(col_ref, row_ref, rank_ref):
    ib = pl.program_id(0)
    si = col_ref[...]
    iglob = ib * IB + lax.broadcasted_iota(jnp.int32, (IB, 128), 0)

    def body(jc, acc):
        sj = row_ref[:, pl.ds(jc * 128, 128)]
        jglob = jc * 128 + lax.broadcasted_iota(jnp.int32, (IB, 128), 1)
        lt = sj < si
        tie = (sj == si) & (jglob < iglob)
        return acc + jnp.where(lt | tie, 1.0, 0.0)

    acc = lax.fori_loop(0, NJ, body, jnp.zeros((IB, 128), jnp.float32))
    rank_ref[...] = jnp.sum(acc, axis=1, keepdims=True).astype(jnp.int32)


def _rank_tc(score_col, score_row):
    return pl.pallas_call(
        _rank_body,
        grid=(NP // IB,),
        in_specs=[
            pl.BlockSpec((IB, 1), lambda i: (i, 0)),
            pl.BlockSpec((1, NP), lambda i: (0, 0)),
        ],
        out_specs=pl.BlockSpec((IB, 1), lambda i: (i, 0)),
        out_shape=jax.ShapeDtypeStruct((NP, 1), jnp.int32),
    )(score_col, score_row)


def _conv_body(sh_ref, w0_ref, w1_ref, w2_ref, b_ref, out_ref):
    sh = sh_ref[...]
    c0 = _dot(sh, w0_ref[...], ((1,), (0,)))
    c1 = _dot(sh, w1_ref[...], ((1,), (0,)))
    c2 = _dot(sh, w2_ref[...], ((1,), (0,)))
    zero = jnp.zeros((1, D), jnp.float32)
    sd = jnp.concatenate([zero, c0[:-1]], axis=0)
    su = jnp.concatenate([c2[1:], zero], axis=0)
    out_ref[...] = c1 + sd + su + b_ref[...]


def _conv_tc(SH, W0, W1c, W2c, cb):
    return pl.pallas_call(
        _conv_body,
        out_shape=jax.ShapeDtypeStruct((NP, D), jnp.float32),
    )(SH, W0, W1c, W2c, cb)


def _phase2_body(r_ref, aggp_ref, w2_ref, b2_ref, seg_ref, feats_ref,
                 wcls_ref, bcls_ref, y_ref, loss_ref, logits_ref):
    rs = r_ref[...] + aggp_ref[0] + aggp_ref[1]
    out = jnp.maximum(_dot(rs, w2_ref[...], ((1,), (0,))) + b2_ref[...], 0.0)

    seg = seg_ref[...]
    gi = lax.broadcasted_iota(jnp.int32, (NP, G), 1)
    onehot = (seg == gi).astype(jnp.float32)
    pooled_sum = _dot(onehot, out, ((0,), (0,)))
    counts = jnp.sum(onehot, axis=0, keepdims=True)
    inv = 1.0 / jnp.maximum(counts, 1.0)
    out_g = pooled_sum * inv.reshape(G, 1)

    lp = _dot(out_g + feats_ref[...], wcls_ref[...], ((1,), (0,))) \
        + bcls_ref[...]
    m = jnp.max(lp, axis=1, keepdims=True)
    ex = jnp.exp(lp - m)
    logits = lp - m - jnp.log(jnp.sum(ex, axis=1, keepdims=True))
    logits_ref[...] = logits

    y = y_ref[...]
    yf = y.astype(jnp.float32)
    n_pos = jnp.maximum(jnp.sum(yf), 1.0)
    n_neg = jnp.maximum(jnp.sum(1.0 - yf), 1.0)
    sw = jnp.where(y == 1, 1.0 / n_pos, 1.0 / n_neg)
    picked = jnp.where(y == 1, logits[:, 1:2], logits[:, 0:1])
    loss_ref[...] = (-jnp.sum(sw * picked) / jnp.sum(sw)).reshape(1, 1)


def _phase2_tc(recnn, agg2p, W2, b2r, seg2, feats, Wcls, bclsr, y2):
    return pl.pallas_call(
        _phase2_body,
        out_shape=[
            jax.ShapeDtypeStruct((1, 1), jnp.float32),
            jax.ShapeDtypeStruct((G, C2), jnp.float32),
        ],
    )(recnn, agg2p, W2, b2r, seg2, feats, Wcls, bclsr, y2)


def kernel(x, edge_index, segment_ids, y, W1, b1, W2, b2, down_k, Wq, Wk, Wv,
           Wo, Wfc, bfc, Wrc, brc, conv_w, conv_b, Wcls, bcls):
    x_ext = jnp.concatenate([x, jnp.zeros((NP - N, D), jnp.float32)], axis=0)
    src = edge_index[0]
    dst = edge_index[1]
    srcp = jnp.concatenate([src, jnp.zeros((EP - E,), jnp.int32)])
    dstp = jnp.concatenate([dst, jnp.full((EP - E,), NP - 1, jnp.int32)])
    seg2 = jnp.concatenate([segment_ids,
                            jnp.full((NP - N,), G, jnp.int32)]).reshape(NP, 1)
    b1r = b1.reshape(1, D)
    b2r = b2.reshape(1, D)
    cbr = conv_b.reshape(1, D)
    bclsr = bcls.reshape(1, C2)
    W0c = conv_w[:, :, 0].T
    W1c = conv_w[:, :, 1].T
    W2c = conv_w[:, :, 2].T
    y2 = y.reshape(G, 1)

    agg = _agg1_sc(x_ext, srcp, dstp)

    h = _h_tc(x + agg[:N], W1, b1r)

    counts0 = jax.ops.segment_sum(jnp.ones((N, 1), jnp.float32),
                                  segment_ids, num_segments=G)
    pooled = jax.ops.segment_sum(h, segment_ids, num_segments=G) \
        / jnp.maximum(counts0, 1.0)
    q = (pooled @ Wq).reshape(G, H, DH)
    kq = (down_k @ Wk).reshape(K, H, DH)
    vq = (down_k @ Wv).reshape(K, H, DH)
    attn = jax.nn.softmax(jnp.einsum('ghd,khd->ghk', q, kq)
                          / jnp.sqrt(float(DH)), axis=-1)
    feats = jnp.einsum('ghk,khd->ghd', attn, vq).reshape(G, D) @ Wo
    z = jnp.tanh(jnp.mean(feats, axis=0))
    feats = feats + ((feats + z) @ Wfc + bfc)
    tail = z @ Wrc + brc
    sc_ = h @ tail
    h_ext = jnp.concatenate([h, jnp.zeros((NP - N, D), jnp.float32)])
    score = jnp.concatenate([sc_, jnp.full((NP - N,), jnp.inf)]).reshape(NP, 1)

    rank = _rank_tc(score, score.reshape(1, NP)).reshape(NP)
    SH = _scatter_rows_sc(h_ext, rank)
    rs2 = _conv_tc(SH, W0c, W1c, W2c, cbr)
    recnn = _gather_rows_sc(rs2, rank)

    agg2p = _edge_agg_sc(recnn, srcp, dstp)
    loss2, logits = _phase2_tc(recnn, agg2p, W2, b2r, seg2, feats, Wcls,
                               bclsr, y2)
    return (loss2.reshape(()), logits)
